# baseline (speedup 1.0000x reference)
import functools

import jax
import jax.numpy as jnp
from jax import lax
from jax.experimental import pallas as pl
from jax.experimental.pallas import tpu as pltpu
from jax.experimental.pallas import tpu_sc as plsc

_VOC = 1000000
_D = 64
_S = 5
_SP = 8
_B = 16384
_NC = 2
_NS = 16
_NW = _NC * _NS

_C = 256
_CSH = 8
_NFULL = _VOC // _C
_TAIL = _VOC - _NFULL * _C
_CPW = _NFULL // _NW
_EXTRA = _NFULL - _NW * _CPW
_NCH = 128
_NBUF = 4
_HCAP = 1968
_SCAP = 80


@functools.lru_cache(maxsize=None)
def _build_sc_gather():
    mesh = plsc.VectorSubcoreMesh(
        core_axis_name="c", subcore_axis_name="s",
        num_cores=_NC, num_subcores=_NS)

    @functools.partial(
        pl.kernel,
        out_type=(
            jax.ShapeDtypeStruct((_B, _D), jnp.float32),
            jax.ShapeDtypeStruct((_B, _D), jnp.float32),
            jax.ShapeDtypeStruct((_SP, _D), jnp.float32),
        ),
        mesh=mesh,
        scratch_types=(
            pltpu.VMEM((_B,), jnp.int32),
            pltpu.VMEM((_B,), jnp.int32),
            pltpu.VMEM((_D, _NBUF * _C), jnp.float32),
            pltpu.VMEM((_HCAP,), jnp.int32),
            pltpu.VMEM((_HCAP,), jnp.int32),
            pltpu.VMEM((_HCAP,), jnp.int32),
            pltpu.VMEM((_HCAP,), jnp.int32),
            pltpu.VMEM((_SCAP, _D), jnp.float32),
            pltpu.VMEM((16,), jnp.int32),
            pltpu.VMEM((_TAIL, _D), jnp.float32),
            pltpu.SMEM((_NCH + 2,), jnp.int32),
            pltpu.SMEM((_NCH + 2,), jnp.int32),
            pltpu.SMEM((_NCH + 2,), jnp.int32),
            pltpu.SemaphoreType.DMA,
            pltpu.SemaphoreType.DMA,
        ),
        compiler_params=pltpu.CompilerParams(needs_layout_passes=False),
    )
    def _sc_gather(ti_hbm, lab_hbm, samp_hbm, tail_hbm, table_hbm,
                   e_out, w_out, sw_out,
                   ti_all, lab_all, cb, hv, hp, sv, sp, stg,
                   samp_v, tail_v, cnts, offs, curs, semc, semr):
        wid = lax.axis_index("s") * _NC + lax.axis_index("c")
        pltpu.sync_copy(ti_hbm, ti_all)
        pltpu.sync_copy(lab_hbm, lab_all)

        start = wid * _CPW
        nfull = jnp.where(wid == _NW - 1, _CPW + _EXTRA, _CPW)
        lo = start * _C
        hi = jnp.where(wid == _NW - 1, _VOC, lo + _CPW * _C)

        iota = lax.iota(jnp.int32, 16)

        def fetch_chunk(cid, half):
            pltpu.async_copy(
                table_hbm.at[:, pl.ds(pl.multiple_of(cid * _C, _C), _C)],
                cb.at[:, pl.ds(pl.multiple_of(half * _C, _C), _C)], semc)

        for b in range(_NBUF):
            @pl.when(b < nfull)
            def _():
                fetch_chunk(start + b, jnp.int32(b))

        def disc(gi, cnt):
            v = ti_all[pl.ds(gi * 16, 16)]
            msk = (v >= lo) & (v < hi)
            plsc.store_compressed(hv.at[pl.ds(cnt, 16)], v, mask=msk)
            plsc.store_compressed(hp.at[pl.ds(cnt, 16)], iota + gi * 16,
                                  mask=msk)
            cnt = cnt + plsc.all_reduce_population_count(msk)[0]
            v = lab_all[pl.ds(gi * 16, 16)]
            msk = (v >= lo) & (v < hi)
            plsc.store_compressed(hv.at[pl.ds(cnt, 16)], v, mask=msk)
            plsc.store_compressed(hp.at[pl.ds(cnt, 16)],
                                  iota + (gi * 16 + _B), mask=msk)
            return cnt + plsc.all_reduce_population_count(msk)[0]

        cnt = lax.fori_loop(0, _B // 16, disc, jnp.int32(0))

        def zero(c, _):
            cnts[c] = 0
            return 0
        lax.fori_loop(0, _NCH + 2, zero, 0)

        ngroups = (cnt + 15) >> 4

        def hist(gi, _):
            c = (hv[pl.ds(gi * 16, 16)] >> _CSH) - start
            for k in range(16):
                ck = jnp.where(gi * 16 + k < cnt, c[k], _NCH)
                cnts[ck] = cnts[ck] + jnp.where(gi * 16 + k < cnt, 1, 0)
            return 0
        lax.fori_loop(0, ngroups, hist, 0)

        def prefix(c, run):
            offs[c] = run
            curs[c] = run
            return run + cnts[c]
        lax.fori_loop(0, _NCH + 2, prefix, jnp.int32(0))

        def scat(gi, _):
            v = hv[pl.ds(gi * 16, 16)]
            p = hp[pl.ds(gi * 16, 16)]
            c = (v >> _CSH) - start
            msk = iota + gi * 16 < cnt
            slots = jnp.zeros((16,), jnp.int32)
            for k in range(16):
                ck = jnp.where(gi * 16 + k < cnt, c[k], _NCH)
                o = curs[ck]
                curs[ck] = o + jnp.where(gi * 16 + k < cnt, 1, 0)
                slots = jnp.where(iota == k, o, slots)
            plsc.store_scatter(sv, [slots], v, mask=msk)
            plsc.store_scatter(sp, [slots], p, mask=msk)
            return 0
        lax.fori_loop(0, ngroups, scat, 0)

        def wait_chunk():
            pltpu.make_async_copy(
                table_hbm.at[:, pl.ds(0, _C)], cb.at[:, pl.ds(0, _C)],
                semc).wait()

        def drain_rows(n):
            def d(_, __):
                pltpu.make_async_copy(
                    stg.at[pl.ds(0, 1)], e_out.at[pl.ds(0, 1)], semr).wait()
                return 0
            lax.fori_loop(0, n, d, 0)

        def fire_row(slot, pos):
            @pl.when(pos < _B)
            def _():
                pltpu.async_copy(stg.at[pl.ds(slot, 1)],
                                 e_out.at[pl.ds(pos, 1)], semr)

            @pl.when(pos >= _B)
            def _():
                pltpu.async_copy(stg.at[pl.ds(slot, 1)],
                                 w_out.at[pl.ds(pos - _B, 1)], semr)

        def process_segment(t, lane_off, clo):
            base = offs[t]
            n = cnts[t]

            def hg(gi, _):
                va = sv[pl.ds(base + gi * 16, 16)]
                pa = sp[pl.ds(base + gi * 16, 16)]
                for k in range(16):
                    @pl.when(gi * 16 + k < n)
                    def _():
                        col = jnp.full((16,), va[k] - clo + lane_off,
                                       jnp.int32)
                        slot = gi * 16 + k
                        for q in range(4):
                            stg[slot, pl.ds(q * 16, 16)] = (
                                plsc.load_gather(cb, [iota + q * 16, col]))
                        fire_row(slot, pa[k])
                return 0

            lax.fori_loop(0, (n + 15) >> 4, hg, 0)
            return n

        def chunk_iter(t, prev):
            half = t & (_NBUF - 1)
            wait_chunk()
            drain_rows(prev)
            n = process_segment(t, half * _C, (start + t) * _C)

            @pl.when(t + _NBUF < nfull)
            def _():
                fetch_chunk(start + t + _NBUF, half)

            return n

        prev = lax.fori_loop(0, nfull, chunk_iter, jnp.int32(0))
        drain_rows(prev)

        @pl.when(wid == _NW - 1)
        def _():
            pltpu.sync_copy(tail_hbm, tail_v)
            t = _CPW + _EXTRA
            base = offs[t]
            n = cnts[t]
            clo = _NFULL * _C

            def hg(gi, _):
                va = sv[pl.ds(base + gi * 16, 16)]
                pa = sp[pl.ds(base + gi * 16, 16)]
                for k in range(16):
                    @pl.when(gi * 16 + k < n)
                    def _():
                        rr = jnp.full((16,), va[k] - clo, jnp.int32)
                        slot = gi * 16 + k
                        for q in range(4):
                            stg[slot, pl.ds(q * 16, 16)] = (
                                plsc.load_gather(tail_v, [rr, iota + q * 16]))
                        fire_row(slot, pa[k])
                return 0

            lax.fori_loop(0, (n + 15) >> 4, hg, 0)
            drain_rows(n)

        @pl.when(wid == 0)
        def _():
            pltpu.sync_copy(samp_hbm, samp_v)
            pltpu.sync_copy(tail_hbm, tail_v)
            svv = samp_v[...]
            for s in range(_S):
                vs = svv[s]

                @pl.when(vs < _NFULL * _C)
                def _():
                    toff = pl.multiple_of(
                        jnp.minimum((vs >> 7) * 128, _NFULL * _C - 256), 128)
                    pltpu.sync_copy(table_hbm.at[:, pl.ds(toff, 256)],
                                    cb.at[:, pl.ds(0, 256)])
                    col = jnp.full((16,), vs - toff, jnp.int32)
                    for q in range(4):
                        stg[s, pl.ds(q * 16, 16)] = (
                            plsc.load_gather(cb, [iota + q * 16, col]))

                @pl.when(vs >= _NFULL * _C)
                def _():
                    rr = jnp.full((16,), vs - _NFULL * _C, jnp.int32)
                    for q in range(4):
                        stg[s, pl.ds(q * 16, 16)] = (
                            plsc.load_gather(tail_v, [rr, iota + q * 16]))

                pltpu.async_copy(stg.at[pl.ds(s, 1)],
                                 sw_out.at[pl.ds(s, 1)], semr)

            def d(_, __):
                pltpu.make_async_copy(
                    stg.at[pl.ds(0, 1)], sw_out.at[pl.ds(0, 1)], semr).wait()
                return 0
            lax.fori_loop(0, _S, d, 0)

    return _sc_gather


_BLK = 2048


def _tc_body(e_ref, w_ref, sw_ref, o_ref):
    e = e_ref[...]
    w = w_ref[...]
    sw = sw_ref[...]
    tl = jnp.sum(e * w, axis=1)
    sl = lax.dot_general(e, sw, (((1,), (1,)), ((), ())),
                         preferred_element_type=jnp.float32)
    col = lax.broadcasted_iota(jnp.int32, sl.shape, 1)
    sl = jnp.where(col < _S, sl, jnp.float32(-1e30))
    m = jnp.maximum(tl, jnp.max(sl, axis=1))
    z = jnp.exp(tl - m) + jnp.sum(jnp.exp(sl - m[:, None]), axis=1)
    o_ref[...] = jnp.log(z) + m - tl


def kernel(train_inputs, labels_inputs, embeddings):
    ti = jnp.squeeze(train_inputs, axis=1)
    lab = jnp.squeeze(labels_inputs, axis=1)
    sampled = jax.random.randint(
        jax.random.key(42), (_S,), 0, _VOC, dtype=jnp.int32)
    samp16 = jnp.concatenate([sampled, jnp.zeros((16 - _S,), jnp.int32)])
    tail = lax.slice(embeddings, (_NFULL * _C, 0), (_VOC, _D))
    e, w, sw = _build_sc_gather()(ti, lab, samp16, tail, embeddings.T)
    loss = pl.pallas_call(
        _tc_body,
        grid=(_B // _BLK,),
        in_specs=[
            pl.BlockSpec((_BLK, _D), lambda i: (i, 0)),
            pl.BlockSpec((_BLK, _D), lambda i: (i, 0)),
            pl.BlockSpec((_SP, _D), lambda i: (0, 0)),
        ],
        out_specs=pl.BlockSpec((_BLK,), lambda i: (i,)),
        out_shape=jax.ShapeDtypeStruct((_B,), jnp.float32),
    )(e, w, sw)
    return loss
